# Initial kernel scaffold; baseline (speedup 1.0000x reference)
import jax
import jax.numpy as jnp
from jax.experimental import pallas as pl


def kernel(node_features, frac_coords, lattice_feats, edge_index, edge2graph, W_e1, b_e1, W_e2, b_e2, W_n1, b_n1, W_n2, b_n2):
    raise NotImplementedError("write your pallas kernel here")


# speedup vs baseline: 3.1746x; 3.1746x over previous
import functools
import jax
import jax.numpy as jnp
from jax import lax
from jax.experimental import pallas as pl
from jax.experimental.pallas import tpu as pltpu
from jax.experimental.pallas import tpu_sc as plsc

F32 = jnp.float32
NW = 32
SUB = 80


def _silu(x):
    return x * jax.nn.sigmoid(x)


def _tc_prep(nfp, fcp, W1a, W1b):
    NP, H = nfp.shape
    BLK = 512
    WID = H + 16

    def body(nf_ref, fc_ref, wa_ref, wb_ref, ts_ref, td_ref):
        nf = nf_ref[...]
        fc = fc_ref[...]
        ts_ref[:, :H] = jnp.dot(nf, wa_ref[...], preferred_element_type=F32)
        ts_ref[:, H:] = -fc
        td_ref[:, :H] = jnp.dot(nf, wb_ref[...], preferred_element_type=F32)
        td_ref[:, H:] = fc

    return pl.pallas_call(
        body,
        grid=(NP // BLK,),
        in_specs=[
            pl.BlockSpec((BLK, H), lambda i: (i, 0)),
            pl.BlockSpec((BLK, 16), lambda i: (i, 0)),
            pl.BlockSpec((H, H), lambda i: (0, 0)),
            pl.BlockSpec((H, H), lambda i: (0, 0)),
        ],
        out_specs=[pl.BlockSpec((BLK, WID), lambda i: (i, 0))] * 2,
        out_shape=[jax.ShapeDtypeStruct((NP, WID), F32)] * 2,
    )(nfp, fcp, W1a, W1b)


def _tc_lat(lat9, W1cp):
    B = lat9.shape[0]

    def body(l_ref, w_ref, out_ref):
        l = l_ref[...]
        cols = []
        for i in range(3):
            for j in range(3):
                v = (l[:, 3 * i + 0:3 * i + 1] * l[:, 3 * j + 0:3 * j + 1]
                     + l[:, 3 * i + 1:3 * i + 2] * l[:, 3 * j + 1:3 * j + 2]
                     + l[:, 3 * i + 2:3 * i + 3] * l[:, 3 * j + 2:3 * j + 3])
                cols.append(v)
        cols.append(jnp.zeros_like(l[:, :7]))
        llt = jnp.concatenate(cols, axis=1)
        out_ref[...] = jnp.dot(llt, w_ref[...], preferred_element_type=F32)

    return pl.pallas_call(
        body,
        out_shape=jax.ShapeDtypeStruct((B, 128), F32),
    )(lat9, W1cp)


def _sc_gather(TS, TD, src2, dst2, E):
    W = TS.shape[1]
    EW = E // NW
    CH = EW // SUB
    mesh = plsc.VectorSubcoreMesh(core_axis_name="c", subcore_axis_name="s")

    @functools.partial(
        pl.kernel,
        mesh=mesh,
        out_type=jax.ShapeDtypeStruct((E, W), F32),
        compiler_params=pltpu.CompilerParams(use_tc_tiling_on_sc=False),
        scratch_types=[
            pltpu.VMEM((CH, SUB), jnp.int32),
            pltpu.VMEM((CH, SUB), jnp.int32),
            pltpu.VMEM((SUB, W), F32),
            pltpu.VMEM((SUB, W), F32),
            pltpu.SemaphoreType.DMA,
            pltpu.SemaphoreType.DMA,
        ],
    )
    def k(ts_hbm, td_hbm, src_hbm, dst_hbm, out_hbm,
          idxs, idxd, bufa, bufb, sema, semb):
        cid = lax.axis_index("c")
        sid = lax.axis_index("s")
        w32 = cid * 16 + sid
        pltpu.sync_copy(src_hbm.at[w32], idxs)
        pltpu.sync_copy(dst_hbm.at[w32], idxd)

        def chunk(j, carry):
            ca = pltpu.async_copy(ts_hbm.at[idxs.at[j]], bufa, sema)
            cb = pltpu.async_copy(td_hbm.at[idxd.at[j]], bufb, semb)
            ca.wait()
            cb.wait()

            def radd(r, c2):
                for cc in range(W // 16):
                    sl = pl.ds(cc * 16, 16)
                    bufa[r, sl] = bufa[r, sl] + bufb[r, sl]
                return c2

            lax.fori_loop(0, SUB, radd, 0)
            pltpu.sync_copy(bufa, out_hbm.at[pl.ds(w32 * EW + j * SUB, SUB)])
            return carry

        lax.fori_loop(0, CH, chunk, 0)

    return k(TS, TD, src2, dst2)


def _tc_edge(G, e2g3, latp, Wfp, W2, b1r, b2r):
    E, WID = G.shape
    H = 128
    B = latp.shape[0]
    BLK = 512

    def body(g_ref, id_ref, lp_ref, wf_ref, w2_ref, b1_ref, b2_ref, out_ref):
        g = g_ref[...]
        d = g[:, H:]
        fd = d - jnp.floor(d)
        ids = id_ref[0, 0, :]
        oh = (ids[:, None] == lax.broadcasted_iota(jnp.int32, (BLK, B), 1)
              ).astype(F32)
        pre = (g[:, :H]
               + jnp.dot(oh, lp_ref[...], preferred_element_type=F32)
               + jnp.dot(fd, wf_ref[...], preferred_element_type=F32)
               + b1_ref[...])
        e1 = _silu(pre)
        out_ref[...] = _silu(
            jnp.dot(e1, w2_ref[...], preferred_element_type=F32) + b2_ref[...])

    return pl.pallas_call(
        body,
        grid=(E // BLK,),
        in_specs=[
            pl.BlockSpec((BLK, WID), lambda i: (i, 0)),
            pl.BlockSpec((1, 1, BLK), lambda i: (i, 0, 0)),
            pl.BlockSpec((B, H), lambda i: (0, 0)),
            pl.BlockSpec((16, H), lambda i: (0, 0)),
            pl.BlockSpec((H, H), lambda i: (0, 0)),
            pl.BlockSpec((1, H), lambda i: (0, 0)),
            pl.BlockSpec((1, H), lambda i: (0, 0)),
        ],
        out_specs=pl.BlockSpec((BLK, H), lambda i: (i, 0)),
        out_shape=jax.ShapeDtypeStruct((E, H), F32),
    )(G, e2g3, latp, Wfp, W2, b1r, b2r)


def _sc_scatter(e2, src2, N2, E):
    H = 128
    EW = E // NW
    CH = EW // SUB
    STRIPE = N2 // 16
    mesh = plsc.VectorSubcoreMesh(core_axis_name="c", subcore_axis_name="s")

    @functools.partial(
        pl.kernel,
        mesh=mesh,
        out_type=(jax.ShapeDtypeStruct((2, N2, H), F32),
                  jax.ShapeDtypeStruct((2, N2, 16), F32)),
        compiler_params=pltpu.CompilerParams(use_tc_tiling_on_sc=False),
        scratch_types=[
            pltpu.VMEM((CH, SUB), jnp.int32),
            pltpu.VMEM((SUB, H), F32),
            pltpu.VMEM((SUB, 16), F32),
            pltpu.VMEM_SHARED((N2, H), F32),
            pltpu.VMEM_SHARED((N2, 16), F32),
        ],
    )
    def k(e2_hbm, src_hbm, sum_out, cnt_out, idxs, ebuf, onesb, acc_sh, cnt_sh):
        cid = lax.axis_index("c")
        sid = lax.axis_index("s")
        w32 = cid * 16 + sid
        pltpu.sync_copy(src_hbm.at[w32], idxs)

        def fz(r, c):
            for cc in range(H // 16):
                ebuf[r, pl.ds(cc * 16, 16)] = jnp.zeros((16,), F32)
            onesb[r, :] = jnp.zeros((16,), F32)
            return c

        lax.fori_loop(0, SUB, fz, 0)

        def zc(t, c):
            pltpu.sync_copy(ebuf, acc_sh.at[pl.ds(sid * STRIPE + t * SUB, SUB)])
            pltpu.sync_copy(onesb, cnt_sh.at[pl.ds(sid * STRIPE + t * SUB, SUB)])
            return c

        lax.fori_loop(0, STRIPE // SUB, zc, 0)

        def fo(r, c):
            onesb[r, :] = jnp.full((16,), 1.0, F32)
            return c

        lax.fori_loop(0, SUB, fo, 0)
        plsc.subcore_barrier()

        def chunk(j, c):
            pltpu.sync_copy(e2_hbm.at[pl.ds(w32 * EW + j * SUB, SUB)], ebuf)
            pltpu.sync_copy(ebuf, acc_sh.at[idxs.at[j]], add=True)
            pltpu.sync_copy(onesb, cnt_sh.at[idxs.at[j]], add=True)
            return c

        lax.fori_loop(0, CH, chunk, 0)
        plsc.subcore_barrier()
        pltpu.sync_copy(acc_sh.at[pl.ds(sid * STRIPE, STRIPE)],
                        sum_out.at[cid, pl.ds(sid * STRIPE, STRIPE)])
        pltpu.sync_copy(cnt_sh.at[pl.ds(sid * STRIPE, STRIPE)],
                        cnt_out.at[cid, pl.ds(sid * STRIPE, STRIPE)])

    return k(e2, src2)


def _tc_node(nfp, p0, p1, c0, c1, Wn1a, Wn1b, Wn2, bn1r, bn2r):
    NP, H = nfp.shape
    BLK = 512

    def body(nf_ref, p0_ref, p1_ref, c0_ref, c1_ref,
             wa_ref, wb_ref, w2_ref, b1_ref, b2_ref, out_ref):
        nf = nf_ref[...]
        cnt = c0_ref[...][:, 0:1] + c1_ref[...][:, 0:1]
        agg = (p0_ref[...] + p1_ref[...]) / jnp.maximum(cnt, 1.0)
        h = (jnp.dot(nf, wa_ref[...], preferred_element_type=F32)
             + jnp.dot(agg, wb_ref[...], preferred_element_type=F32)
             + b1_ref[...])
        o = _silu(h)
        out_ref[...] = nf + _silu(
            jnp.dot(o, w2_ref[...], preferred_element_type=F32) + b2_ref[...])

    return pl.pallas_call(
        body,
        grid=(NP // BLK,),
        in_specs=[
            pl.BlockSpec((BLK, H), lambda i: (i, 0)),
            pl.BlockSpec((BLK, H), lambda i: (i, 0)),
            pl.BlockSpec((BLK, H), lambda i: (i, 0)),
            pl.BlockSpec((BLK, 16), lambda i: (i, 0)),
            pl.BlockSpec((BLK, 16), lambda i: (i, 0)),
            pl.BlockSpec((H, H), lambda i: (0, 0)),
            pl.BlockSpec((H, H), lambda i: (0, 0)),
            pl.BlockSpec((H, H), lambda i: (0, 0)),
            pl.BlockSpec((1, H), lambda i: (0, 0)),
            pl.BlockSpec((1, H), lambda i: (0, 0)),
        ],
        out_specs=pl.BlockSpec((BLK, H), lambda i: (i, 0)),
        out_shape=jax.ShapeDtypeStruct((NP, H), F32),
    )(nfp, p0, p1, c0, c1, Wn1a, Wn1b, Wn2, bn1r, bn2r)


def kernel(node_features, frac_coords, lattice_feats, edge_index, edge2graph,
           W_e1, b_e1, W_e2, b_e2, W_n1, b_n1, W_n2, b_n2):
    N, H = node_features.shape
    E = edge_index.shape[1]
    B = lattice_feats.shape[0]
    NP = ((N + 511) // 512) * 512

    nfp = jnp.pad(node_features, ((0, NP - N), (0, 0)))
    fcp = jnp.pad(frac_coords, ((0, NP - N), (0, 13)))
    W1a = W_e1[:H]
    W1b = W_e1[H:2 * H]
    W1cp = jnp.pad(W_e1[2 * H:2 * H + 9], ((0, 7), (0, 0)))
    Wfp = jnp.pad(W_e1[2 * H + 9:], ((0, 13), (0, 0)))
    lat9 = jnp.pad(lattice_feats.reshape(B, 9), ((0, 0), (0, 7)))

    TS, TD = _tc_prep(nfp, fcp, W1a, W1b)
    latp = _tc_lat(lat9, W1cp)

    CH = E // NW // SUB
    src2 = edge_index[0].reshape(NW, CH, SUB)
    dst2 = edge_index[1].reshape(NW, CH, SUB)
    G = _sc_gather(TS, TD, src2, dst2, E)

    e2g3 = edge2graph.reshape(E // 512, 1, 512)
    e2 = _tc_edge(G, e2g3, latp, Wfp, W_e2,
                  b_e1.reshape(1, H), b_e2.reshape(1, H))

    sums, cnts = _sc_scatter(e2, src2, NP, E)

    out = _tc_node(nfp, sums[0], sums[1], cnts[0], cnts[1],
                   W_n1[:H], W_n1[H:], W_n2,
                   b_n1.reshape(1, H), b_n2.reshape(1, H))
    return out[:N]
